# baseline (speedup 1.0000x reference)
import functools

import jax
import jax.numpy as jnp
from jax import lax
from jax.experimental import pallas as pl
from jax.experimental.pallas import tpu as pltpu
from jax.experimental.pallas import tpu_sc as plsc

S_N = 2048
A_N = 16
R_N = 2
B_SZ = 4096
N_T = 204800

NC = 2
NS = 16
NW = NC * NS
PER_W = N_T // NW
CH = 128
NCH = PER_W // CH

BLK_S = 32


def _tables_body(sa_ref, a_ref, r_ref, c_ref, w_ref):
    x = sa_ref[...]
    m = jnp.max(x, axis=-1)
    e = jnp.exp(x - m[:, :, None])
    lse_sa = m + jnp.log(jnp.sum(e, axis=-1))

    av = a_ref[...]
    am = jnp.max(av, axis=-1, keepdims=True)
    lse_a = am + jnp.log(jnp.sum(jnp.exp(av - am), axis=-1, keepdims=True))
    rv = r_ref[...]
    rm = jnp.max(rv, axis=-1, keepdims=True)
    lse_r = rm + jnp.log(jnp.sum(jnp.exp(rv - rm), axis=-1, keepdims=True))

    base_a = av - lse_a
    br0 = rv[:, 0:1] - lse_r
    br1 = rv[:, 1:2] - lse_r
    c_ref[...] = jnp.concatenate(
        [base_a + br0 - lse_sa,
         base_a + br0,
         base_a + br1 - lse_sa,
         base_a + br1,
         jnp.zeros((BLK_S, 64), jnp.float32)],
        axis=-1)

    t = lax.bitcast_convert_type(x, jnp.int32)
    words = (((t[:, :, 1024:] + 32768) & (-65536))
             | (((t[:, :, :1024] + 32768) >> 16) & 65535))
    rows = words.reshape(BLK_S * A_N, 1024)
    for tc in range(8):
        w_ref[:, tc, :, :] = rows[:, tc * 128:(tc + 1) * 128].reshape(
            BLK_S * A_N // 8, 8, 128)


def _build_tables(params_s_sa, params_a_s, params_r_s):
    grid = (S_N // BLK_S,)
    return pl.pallas_call(
        _tables_body,
        grid=grid,
        in_specs=[
            pl.BlockSpec((BLK_S, A_N, S_N), lambda i: (i, 0, 0)),
            pl.BlockSpec((BLK_S, A_N), lambda i: (i, 0)),
            pl.BlockSpec((BLK_S, R_N), lambda i: (i, 0)),
        ],
        out_specs=[
            pl.BlockSpec((BLK_S, 128), lambda i: (i, 0)),
            pl.BlockSpec((BLK_S * A_N // 8, 8, 8, 128), lambda i: (i, 0, 0, 0)),
        ],
        out_shape=[
            jax.ShapeDtypeStruct((S_N, 128), jnp.float32),
            jax.ShapeDtypeStruct((S_N * A_N // 8, 8, 8, 128), jnp.int32),
        ],
    )(params_s_sa, params_a_s, params_r_s)


def _sc_body(ctab_hbm, w_hbm, s_hbm, a_hbm, r_hbm, d_hbm, sn_hbm, out_hbm,
             s_v, a_v, r_v, d_v, sn_v, idx1_v, idx2_v, val1_v, val2_v,
             acc_v, sem1, sem2):
    wid = lax.axis_index("sub") * NC + lax.axis_index("core")
    base = wid * PER_W

    pltpu.sync_copy(s_hbm.at[pl.ds(base, PER_W)], s_v)
    pltpu.sync_copy(a_hbm.at[pl.ds(base, PER_W)], a_v)
    pltpu.sync_copy(r_hbm.at[pl.ds(base, PER_W)], r_v)
    pltpu.sync_copy(d_hbm.at[pl.ds(base, PER_W)], d_v)
    pltpu.sync_copy(sn_hbm.at[pl.ds(base, PER_W)], sn_v)

    def idx_body(i, _):
        off = i * 16
        sv = s_v[pl.ds(off, 16)]
        avv = a_v[pl.ds(off, 16)]
        rvv = r_v[pl.ds(off, 16)]
        dv = d_v[pl.ds(off, 16)]
        snv = sn_v[pl.ds(off, 16)]
        r2 = sv * 16 + avv
        cw = snv & 1023
        idx1_v[pl.ds(off, 16)] = (sv << 7) | ((rvv * 2 + dv) << 4) | avv
        idx2_v[pl.ds(off, 16)] = (((r2 >> 3) << 13) | ((cw >> 7) << 10)
                                  | ((r2 & 7) << 7) | (cw & 127))
        return 0
    lax.fori_loop(0, PER_W // 16, idx_body, 0)

    def fire_body(j, _):
        off = j * CH
        pltpu.async_copy(ctab_hbm.at[idx1_v.at[pl.ds(off, CH)]],
                         val1_v.at[pl.ds(off, CH)], sem1)
        pltpu.async_copy(w_hbm.at[idx2_v.at[pl.ds(off, CH)]],
                         val2_v.at[pl.ds(off, CH)], sem2)
        return 0
    lax.fori_loop(0, NCH, fire_body, 0)

    def acc1_body(i, acc):
        return acc + val1_v[pl.ds(i * 16, 16)]

    pltpu.make_async_copy(ctab_hbm.at[pl.ds(0, PER_W)], val1_v, sem1).wait()
    acc = lax.fori_loop(0, PER_W // 16, acc1_body,
                        jnp.zeros((16,), jnp.float32))

    pltpu.make_async_copy(w_hbm.at[pl.ds(0, PER_W)], val2_v, sem2).wait()

    def acc2_body(i, acc):
        off = i * 16
        dv = d_v[pl.ds(off, 16)]
        snv = sn_v[pl.ds(off, 16)]
        wv = val2_v[pl.ds(off, 16)]
        sel = snv >> 10
        bits = jnp.where(sel == 0, wv << 16, wv & (-65536))
        v2 = lax.bitcast_convert_type(bits, jnp.float32)
        return acc + jnp.where(dv == 0, v2, 0.0)
    acc = lax.fori_loop(0, PER_W // 16, acc2_body, acc)
    acc_v[...] = acc
    pltpu.sync_copy(acc_v, out_hbm.at[wid])


_SC_CACHE = {}


def _make_sc_gather():
    if "k" not in _SC_CACHE:
        _SC_CACHE["k"] = functools.partial(
            pl.kernel,
            out_type=jax.ShapeDtypeStruct((NW, 16), jnp.float32),
            mesh=plsc.VectorSubcoreMesh(core_axis_name="core",
                                        subcore_axis_name="sub",
                                        num_cores=NC, num_subcores=NS),
            scratch_types=[
                pltpu.VMEM((PER_W,), jnp.int32),
                pltpu.VMEM((PER_W,), jnp.int32),
                pltpu.VMEM((PER_W,), jnp.int32),
                pltpu.VMEM((PER_W,), jnp.int32),
                pltpu.VMEM((PER_W,), jnp.int32),
                pltpu.VMEM((PER_W,), jnp.int32),
                pltpu.VMEM((PER_W,), jnp.int32),
                pltpu.VMEM((PER_W,), jnp.float32),
                pltpu.VMEM((PER_W,), jnp.int32),
                pltpu.VMEM((16,), jnp.float32),
                pltpu.SemaphoreType.DMA,
                pltpu.SemaphoreType.DMA,
            ],
            name="nll_gather_sc",
        )(_sc_body)
    return _SC_CACHE["k"]


def kernel(params_s, params_s_sa, params_o_s, params_r_s, params_a_s,
           regime, s, a, r_cat, d, s_next):
    ctab, wtab = _build_tables(params_s_sa, params_a_s, params_r_s)
    ctab_flat = ctab.reshape(-1)
    w_flat = wtab.reshape(-1)
    partials = _make_sc_gather()(ctab_flat, w_flat, s, a, r_cat, d, s_next)
    return -jnp.sum(partials) / B_SZ


# speedup vs baseline: 1.0078x; 1.0078x over previous
import functools

import jax
import jax.numpy as jnp
from jax import lax
from jax.experimental import pallas as pl
from jax.experimental.pallas import tpu as pltpu
from jax.experimental.pallas import tpu_sc as plsc

S_N = 2048
A_N = 16
R_N = 2
B_SZ = 4096
N_T = 204800

NC = 2
NS = 16
NW = NC * NS
PER_W = N_T // NW
CH = 128
NCH = PER_W // CH

BLK_S = 32


def _tables_body(sa_ref, a_ref, r_ref, c_ref, w_ref):
    x = sa_ref[...]
    m = jnp.max(x, axis=-1)
    e = jnp.exp(x - m[:, :, None])
    lse_sa = m + jnp.log(jnp.sum(e, axis=-1))

    av = a_ref[...]
    am = jnp.max(av, axis=-1, keepdims=True)
    lse_a = am + jnp.log(jnp.sum(jnp.exp(av - am), axis=-1, keepdims=True))
    rv = r_ref[...]
    rm = jnp.max(rv, axis=-1, keepdims=True)
    lse_r = rm + jnp.log(jnp.sum(jnp.exp(rv - rm), axis=-1, keepdims=True))

    base_a = av - lse_a
    br0 = rv[:, 0:1] - lse_r
    br1 = rv[:, 1:2] - lse_r
    c_ref[...] = jnp.concatenate(
        [base_a + br0 - lse_sa,
         base_a + br0,
         base_a + br1 - lse_sa,
         base_a + br1,
         jnp.zeros((BLK_S, 64), jnp.float32)],
        axis=-1)

    t = lax.bitcast_convert_type(x, jnp.int32)
    words = (((t[:, :, 1024:] + 32768) & (-65536))
             | (((t[:, :, :1024] + 32768) >> 16) & 65535))
    rows = words.reshape(BLK_S * A_N, 1024)
    for tc in range(8):
        w_ref[:, tc, :, :] = rows[:, tc * 128:(tc + 1) * 128].reshape(
            BLK_S * A_N // 8, 8, 128)


def _build_tables(params_s_sa, params_a_s, params_r_s):
    grid = (S_N // BLK_S,)
    return pl.pallas_call(
        _tables_body,
        grid=grid,
        in_specs=[
            pl.BlockSpec((BLK_S, A_N, S_N), lambda i: (i, 0, 0)),
            pl.BlockSpec((BLK_S, A_N), lambda i: (i, 0)),
            pl.BlockSpec((BLK_S, R_N), lambda i: (i, 0)),
        ],
        out_specs=[
            pl.BlockSpec((BLK_S, 128), lambda i: (i, 0)),
            pl.BlockSpec((BLK_S * A_N // 8, 8, 8, 128), lambda i: (i, 0, 0, 0)),
        ],
        out_shape=[
            jax.ShapeDtypeStruct((S_N, 128), jnp.float32),
            jax.ShapeDtypeStruct((S_N * A_N // 8, 8, 8, 128), jnp.int32),
        ],
    )(params_s_sa, params_a_s, params_r_s)


def _sc_body(ctab_hbm, w_hbm, s_hbm, a_hbm, r_hbm, d_hbm, sn_hbm, out_hbm,
             s_v, a_v, r_v, d_v, sn_v, idx1_v, idx2_v, val1_v, val2_v,
             acc_v, sem1, sem2):
    wid = lax.axis_index("sub") * NC + lax.axis_index("core")
    base = wid * PER_W

    pltpu.sync_copy(s_hbm.at[pl.ds(base, PER_W)], s_v)
    pltpu.sync_copy(a_hbm.at[pl.ds(base, PER_W)], a_v)
    pltpu.sync_copy(r_hbm.at[pl.ds(base, PER_W)], r_v)
    pltpu.sync_copy(d_hbm.at[pl.ds(base, PER_W)], d_v)
    pltpu.sync_copy(sn_hbm.at[pl.ds(base, PER_W)], sn_v)

    def idx_body(i, _):
        off = i * 16
        sv = s_v[pl.ds(off, 16)]
        avv = a_v[pl.ds(off, 16)]
        rvv = r_v[pl.ds(off, 16)]
        dv = d_v[pl.ds(off, 16)]
        snv = sn_v[pl.ds(off, 16)]
        r2 = sv * 16 + avv
        cw = snv & 1023
        idx1_v[pl.ds(off, 16)] = (sv << 7) | ((rvv * 2 + dv) << 4) | avv
        idx2_v[pl.ds(off, 16)] = (((r2 >> 3) << 13) | ((cw >> 7) << 10)
                                  | ((r2 & 7) << 7) | (cw & 127))
        return 0
    lax.fori_loop(0, PER_W // 16, idx_body, 0)

    def fire_body(j, _):
        off = j * CH
        pltpu.async_copy(ctab_hbm.at[idx1_v.at[pl.ds(off, CH)]],
                         val1_v.at[pl.ds(off, CH)], sem1)
        pltpu.async_copy(w_hbm.at[idx2_v.at[pl.ds(off, CH)]],
                         val2_v.at[pl.ds(off, CH)], sem2)
        return 0
    lax.fori_loop(0, NCH, fire_body, 0)

    pltpu.make_async_copy(ctab_hbm.at[pl.ds(0, PER_W)], val1_v, sem1).wait()
    pltpu.make_async_copy(w_hbm.at[pl.ds(0, PER_W)], val2_v, sem2).wait()

    def acc_body(i, acc):
        off = i * 16
        dv = d_v[pl.ds(off, 16)]
        snv = sn_v[pl.ds(off, 16)]
        wv = val2_v[pl.ds(off, 16)]
        v1 = val1_v[pl.ds(off, 16)]
        sel = snv >> 10
        bits = jnp.where(sel == 0, wv << 16, wv & (-65536))
        v2 = lax.bitcast_convert_type(bits, jnp.float32)
        return acc + v1 + jnp.where(dv == 0, v2, 0.0)
    acc = lax.fori_loop(0, PER_W // 16, acc_body,
                        jnp.zeros((16,), jnp.float32))
    acc_v[...] = acc
    pltpu.sync_copy(acc_v, out_hbm.at[wid])


_SC_CACHE = {}


def _make_sc_gather():
    if "k" not in _SC_CACHE:
        _SC_CACHE["k"] = functools.partial(
            pl.kernel,
            out_type=jax.ShapeDtypeStruct((NW, 16), jnp.float32),
            mesh=plsc.VectorSubcoreMesh(core_axis_name="core",
                                        subcore_axis_name="sub",
                                        num_cores=NC, num_subcores=NS),
            scratch_types=[
                pltpu.VMEM((PER_W,), jnp.int32),
                pltpu.VMEM((PER_W,), jnp.int32),
                pltpu.VMEM((PER_W,), jnp.int32),
                pltpu.VMEM((PER_W,), jnp.int32),
                pltpu.VMEM((PER_W,), jnp.int32),
                pltpu.VMEM((PER_W,), jnp.int32),
                pltpu.VMEM((PER_W,), jnp.int32),
                pltpu.VMEM((PER_W,), jnp.float32),
                pltpu.VMEM((PER_W,), jnp.int32),
                pltpu.VMEM((16,), jnp.float32),
                pltpu.SemaphoreType.DMA,
                pltpu.SemaphoreType.DMA,
            ],
            name="nll_gather_sc",
        )(_sc_body)
    return _SC_CACHE["k"]


def kernel(params_s, params_s_sa, params_o_s, params_r_s, params_a_s,
           regime, s, a, r_cat, d, s_next):
    ctab, wtab = _build_tables(params_s_sa, params_a_s, params_r_s)
    ctab_flat = ctab.reshape(-1)
    w_flat = wtab.reshape(-1)
    partials = _make_sc_gather()(ctab_flat, w_flat, s, a, r_cat, d, s_next)
    return -jnp.sum(partials) / B_SZ
